# Initial kernel scaffold; baseline (speedup 1.0000x reference)
import jax
import jax.numpy as jnp
from jax.experimental import pallas as pl


def kernel(scores):
    raise NotImplementedError("write your pallas kernel here")


# speedup vs baseline: 35.4776x; 35.4776x over previous
import functools

import jax
import jax.numpy as jnp
from jax import lax
from jax.experimental import pallas as pl
from jax.experimental.pallas import tpu as pltpu
from jax.experimental.pallas import tpu_sc as plsc

NUM_TOKENS = 32768
NUM_EXPERTS = 256
N_GROUP = 8
EPG = NUM_EXPERTS // N_GROUP
TOPK_GROUP = 4

NC, NS, L = 2, 16, 16
NW = NC * NS
TOK_PER_W = NUM_TOKENS // NW
CHUNK = 64
NCHUNK = TOK_PER_W // CHUNK
SUB = CHUNK // L

_mesh = plsc.VectorSubcoreMesh(
    core_axis_name="c", subcore_axis_name="s", num_cores=NC, num_subcores=NS
)


@functools.partial(
    pl.kernel,
    out_type=(
        jax.ShapeDtypeStruct((NUM_TOKENS, NUM_EXPERTS), jnp.float32),
        jax.ShapeDtypeStruct((NUM_TOKENS, N_GROUP), jnp.float32),
    ),
    mesh=_mesh,
    scratch_types=[
        pltpu.VMEM((CHUNK, NUM_EXPERTS), jnp.float32),
        pltpu.VMEM((CHUNK, NUM_EXPERTS), jnp.float32),
        pltpu.VMEM((CHUNK, N_GROUP), jnp.float32),
    ],
    compiler_params=pltpu.CompilerParams(
        use_tc_tiling_on_sc=False, needs_layout_passes=False
    ),
)
def _moe_route(scores_hbm, masked_hbm, gmask_hbm, in_v, out_v, gm_v):
    wid = lax.axis_index("s") * NC + lax.axis_index("c")
    tok_base = wid * TOK_PER_W
    lanes = lax.iota(jnp.int32, L)
    neg_inf = jnp.full((L,), -jnp.inf, jnp.float32)

    def chunk_body(ci, carry):
        tok0 = tok_base + ci * CHUNK
        pltpu.sync_copy(scores_hbm.at[pl.ds(tok0, CHUNK)], in_v)

        def sub_body(s, carry2):
            row = lanes + s * L
            m1 = [neg_inf] * N_GROUP
            m2 = [neg_inf] * N_GROUP
            for e in range(NUM_EXPERTS):
                g = e // EPG
                col = jnp.full((L,), e, jnp.int32)
                x = plsc.load_gather(in_v, [row, col])
                t = jnp.minimum(m1[g], x)
                m1[g] = jnp.maximum(m1[g], x)
                m2[g] = jnp.maximum(m2[g], t)
            gs = [m1[g] + m2[g] for g in range(N_GROUP)]
            rank = [jnp.zeros((L,), jnp.int32) for _ in range(N_GROUP)]
            one = jnp.ones((L,), jnp.int32)
            zero = jnp.zeros((L,), jnp.int32)
            for g in range(N_GROUP):
                for h in range(g + 1, N_GROUP):
                    ge = gs[g] >= gs[h]
                    rank[h] = rank[h] + jnp.where(ge, one, zero)
                    rank[g] = rank[g] + jnp.where(ge, zero, one)
            sel = [rank[g] < TOPK_GROUP for g in range(N_GROUP)]
            fone = jnp.ones((L,), jnp.float32)
            fzero = jnp.zeros((L,), jnp.float32)
            for g in range(N_GROUP):
                gcol = jnp.full((L,), g, jnp.int32)
                plsc.store_scatter(gm_v, [row, gcol], jnp.where(sel[g], fone, fzero))
            for e in range(NUM_EXPERTS):
                g = e // EPG
                col = jnp.full((L,), e, jnp.int32)
                x = plsc.load_gather(in_v, [row, col])
                plsc.store_scatter(out_v, [row, col], jnp.where(sel[g], x, neg_inf))
            return carry2

        lax.fori_loop(0, SUB, sub_body, 0)
        pltpu.sync_copy(out_v, masked_hbm.at[pl.ds(tok0, CHUNK)])
        pltpu.sync_copy(gm_v, gmask_hbm.at[pl.ds(tok0, CHUNK)])
        return carry

    lax.fori_loop(0, NCHUNK, chunk_body, 0)


def kernel(scores):
    return _moe_route(scores)


# speedup vs baseline: 57.2870x; 1.6147x over previous
import functools

import jax
import jax.numpy as jnp
from jax import lax
from jax.experimental import pallas as pl
from jax.experimental.pallas import tpu as pltpu
from jax.experimental.pallas import tpu_sc as plsc

NUM_TOKENS = 32768
NUM_EXPERTS = 256
N_GROUP = 8
EPG = NUM_EXPERTS // N_GROUP
TOPK_GROUP = 4

NC, NS, L = 2, 16, 16
NW = NC * NS
TOK_PER_W = NUM_TOKENS // NW
CHUNK = 64
NCHUNK = TOK_PER_W // CHUNK
SUB = CHUNK // L

_mesh = plsc.VectorSubcoreMesh(
    core_axis_name="c", subcore_axis_name="s", num_cores=NC, num_subcores=NS
)


@functools.partial(
    pl.kernel,
    out_type=(
        jax.ShapeDtypeStruct((NUM_TOKENS, NUM_EXPERTS), jnp.float32),
        jax.ShapeDtypeStruct((NUM_TOKENS, N_GROUP), jnp.float32),
    ),
    mesh=_mesh,
    scratch_types=[
        pltpu.VMEM((CHUNK, NUM_EXPERTS), jnp.float32),
        pltpu.VMEM((CHUNK, NUM_EXPERTS), jnp.float32),
        pltpu.VMEM((CHUNK, N_GROUP), jnp.float32),
    ],
    compiler_params=pltpu.CompilerParams(
        use_tc_tiling_on_sc=False, needs_layout_passes=False
    ),
)
def _moe_route(scores_hbm, masked_hbm, gmask_hbm, in_v, out_v, gm_v):
    wid = lax.axis_index("s") * NC + lax.axis_index("c")
    tok_base = wid * TOK_PER_W
    lanes = lax.iota(jnp.int32, L)
    neg_inf = jnp.full((L,), -jnp.inf, jnp.float32)
    rot = [(lanes + k) & (L - 1) for k in range(L)]

    def chunk_body(ci, carry):
        tok0 = tok_base + ci * CHUNK
        pltpu.sync_copy(scores_hbm.at[pl.ds(tok0, CHUNK)], in_v)

        def sub_body(s, carry2):
            row = lanes + s * L
            m1 = [neg_inf] * N_GROUP
            m2 = [neg_inf] * N_GROUP
            for tile in range(NUM_EXPERTS // L):
                g = (tile * L) // EPG
                for k in range(L):
                    col = rot[k] + tile * L
                    x = plsc.load_gather(in_v, [row, col])
                    t = jnp.minimum(m1[g], x)
                    m1[g] = jnp.maximum(m1[g], x)
                    m2[g] = jnp.maximum(m2[g], t)
            gs = [m1[g] + m2[g] for g in range(N_GROUP)]
            rank = [jnp.zeros((L,), jnp.int32) for _ in range(N_GROUP)]
            one = jnp.ones((L,), jnp.int32)
            zero = jnp.zeros((L,), jnp.int32)
            for g in range(N_GROUP):
                for h in range(g + 1, N_GROUP):
                    ge = gs[g] >= gs[h]
                    rank[h] = rank[h] + jnp.where(ge, one, zero)
                    rank[g] = rank[g] + jnp.where(ge, zero, one)
            sel = [rank[g] < TOPK_GROUP for g in range(N_GROUP)]
            fone = jnp.ones((L,), jnp.float32)
            fzero = jnp.zeros((L,), jnp.float32)
            for g in range(N_GROUP):
                gcol = jnp.full((L,), g, jnp.int32)
                plsc.store_scatter(gm_v, [row, gcol], jnp.where(sel[g], fone, fzero))
            for tile in range(NUM_EXPERTS // L):
                g = (tile * L) // EPG
                for k in range(L):
                    col = rot[k] + tile * L
                    x = plsc.load_gather(in_v, [row, col])
                    plsc.store_scatter(out_v, [row, col], jnp.where(sel[g], x, neg_inf))
            return carry2

        lax.fori_loop(0, SUB, sub_body, 0)
        pltpu.sync_copy(out_v, masked_hbm.at[pl.ds(tok0, CHUNK)])
        pltpu.sync_copy(gm_v, gmask_hbm.at[pl.ds(tok0, CHUNK)])
        return carry

    lax.fori_loop(0, NCHUNK, chunk_body, 0)


def kernel(scores):
    return _moe_route(scores)


# speedup vs baseline: 129.1106x; 2.2537x over previous
import functools

import jax
import jax.numpy as jnp
from jax import lax
from jax.experimental import pallas as pl
from jax.experimental.pallas import tpu as pltpu
from jax.experimental.pallas import tpu_sc as plsc

NUM_TOKENS = 32768
NUM_EXPERTS = 256
N_GROUP = 8
EPG = NUM_EXPERTS // N_GROUP
TOPK_GROUP = 4

NC, NS, L = 2, 16, 16
NW = NC * NS
TOK_PER_W = NUM_TOKENS // NW
CHUNK = 128
NCHUNK = TOK_PER_W // CHUNK
NPAIR = NCHUNK // 2
SUB = CHUNK // L

_mesh = plsc.VectorSubcoreMesh(
    core_axis_name="c", subcore_axis_name="s", num_cores=NC, num_subcores=NS
)


@functools.partial(
    pl.kernel,
    out_type=(
        jax.ShapeDtypeStruct((NUM_TOKENS, NUM_EXPERTS), jnp.float32),
        jax.ShapeDtypeStruct((NUM_TOKENS, N_GROUP), jnp.float32),
    ),
    mesh=_mesh,
    scratch_types=[
        pltpu.VMEM((2 * CHUNK, NUM_EXPERTS), jnp.float32),
        pltpu.VMEM((2 * CHUNK, N_GROUP), jnp.float32),
        pltpu.SemaphoreType.DMA,
        pltpu.SemaphoreType.DMA,
        pltpu.SemaphoreType.DMA,
        pltpu.SemaphoreType.DMA,
    ],
    compiler_params=pltpu.CompilerParams(
        use_tc_tiling_on_sc=False, needs_layout_passes=False
    ),
)
def _moe_route(scores_hbm, masked_hbm, gmask_hbm, buf, gm_v, si_a, si_b, so_a, so_b):
    wid = lax.axis_index("s") * NC + lax.axis_index("c")
    tok_base = wid * TOK_PER_W
    lanes = lax.iota(jnp.int32, L)
    neg_inf = jnp.full((L,), -jnp.inf, jnp.float32)
    rot = [(lanes + k) & (L - 1) for k in range(L)]

    in_sems = (si_a, si_b)
    out_sems = (so_a, so_b)

    def start_in(ci, slot):
        pltpu.async_copy(
            scores_hbm.at[pl.ds(tok_base + ci * CHUNK, CHUNK)],
            buf.at[pl.ds(slot * CHUNK, CHUNK)],
            in_sems[slot],
        )

    def wait_in(slot):
        pltpu.make_async_copy(
            scores_hbm.at[pl.ds(0, CHUNK)],
            buf.at[pl.ds(slot * CHUNK, CHUNK)],
            in_sems[slot],
        ).wait()

    def start_out(ci, slot):
        pltpu.async_copy(
            buf.at[pl.ds(slot * CHUNK, CHUNK)],
            masked_hbm.at[pl.ds(tok_base + ci * CHUNK, CHUNK)],
            out_sems[slot],
        )
        pltpu.async_copy(
            gm_v.at[pl.ds(slot * CHUNK, CHUNK)],
            gmask_hbm.at[pl.ds(tok_base + ci * CHUNK, CHUNK)],
            out_sems[slot],
        )

    def wait_out(slot):
        pltpu.make_async_copy(
            buf.at[pl.ds(slot * CHUNK, CHUNK)],
            masked_hbm.at[pl.ds(0, CHUNK)],
            out_sems[slot],
        ).wait()
        pltpu.make_async_copy(
            gm_v.at[pl.ds(slot * CHUNK, CHUNK)],
            gmask_hbm.at[pl.ds(0, CHUNK)],
            out_sems[slot],
        ).wait()

    def compute(slot):
        def sub_body(s, carry2):
            row = lanes + (s * L + slot * CHUNK)
            m1 = [neg_inf] * N_GROUP
            m2 = [neg_inf] * N_GROUP
            for tile in range(NUM_EXPERTS // L):
                g = (tile * L) // EPG
                for k in range(L):
                    col = rot[k] + tile * L
                    x = plsc.load_gather(buf, [row, col])
                    t = jnp.minimum(m1[g], x)
                    m1[g] = jnp.maximum(m1[g], x)
                    m2[g] = jnp.maximum(m2[g], t)
            gs = [m1[g] + m2[g] for g in range(N_GROUP)]
            rank = [jnp.zeros((L,), jnp.int32) for _ in range(N_GROUP)]
            one = jnp.ones((L,), jnp.int32)
            zero = jnp.zeros((L,), jnp.int32)
            for g in range(N_GROUP):
                for h in range(g + 1, N_GROUP):
                    ge = gs[g] >= gs[h]
                    rank[h] = rank[h] + jnp.where(ge, one, zero)
                    rank[g] = rank[g] + jnp.where(ge, zero, one)
            sel = [rank[g] < TOPK_GROUP for g in range(N_GROUP)]
            fone = jnp.ones((L,), jnp.float32)
            fzero = jnp.zeros((L,), jnp.float32)
            for g in range(N_GROUP):
                gcol = jnp.full((L,), g, jnp.int32)
                plsc.store_scatter(gm_v, [row, gcol], jnp.where(sel[g], fone, fzero))
            for tile in range(NUM_EXPERTS // L):
                g = (tile * L) // EPG
                nsel = rank[g] >= TOPK_GROUP
                for k in range(L):
                    col = rot[k] + tile * L
                    plsc.store_scatter(buf, [row, col], neg_inf, mask=nsel)
            return carry2

        lax.fori_loop(0, SUB, sub_body, 0)

    start_in(0, 0)

    def pair_body(i, carry):
        c0 = 2 * i

        @pl.when(i > 0)
        def _():
            wait_out(1)

        start_in(c0 + 1, 1)
        wait_in(0)
        compute(0)
        start_out(c0, 0)

        wait_in(1)
        compute(1)

        @pl.when(i < NPAIR - 1)
        def _():
            wait_out(0)
            start_in(c0 + 2, 0)

        start_out(c0 + 1, 1)
        return carry

    lax.fori_loop(0, NPAIR, pair_body, 0)
    wait_out(0)
    wait_out(1)


def kernel(scores):
    return _moe_route(scores)


# speedup vs baseline: 131.6395x; 1.0196x over previous
import functools

import jax
import jax.numpy as jnp
from jax import lax
from jax.experimental import pallas as pl
from jax.experimental.pallas import tpu as pltpu
from jax.experimental.pallas import tpu_sc as plsc

NUM_TOKENS = 32768
NUM_EXPERTS = 256
N_GROUP = 8
EPG = NUM_EXPERTS // N_GROUP
TOPK_GROUP = 4

NC, NS, L = 2, 16, 16
NW = NC * NS
TOK_PER_W = NUM_TOKENS // NW
CHUNK = 128
RPC = CHUNK // 8
NCHUNK = TOK_PER_W // CHUNK
NPAIR = NCHUNK // 2
SUB = CHUNK // L
NROWS = NUM_TOKENS // 8

_mesh = plsc.VectorSubcoreMesh(
    core_axis_name="c", subcore_axis_name="s", num_cores=NC, num_subcores=NS
)


@functools.partial(
    pl.kernel,
    out_type=(
        jax.ShapeDtypeStruct((NROWS, 16, 128), jnp.float32),
        jax.ShapeDtypeStruct((NUM_TOKENS // 128, N_GROUP, 128), jnp.float32),
    ),
    mesh=_mesh,
    scratch_types=[
        pltpu.VMEM((2 * RPC, 16, 128), jnp.float32),
        pltpu.VMEM((2, N_GROUP, CHUNK), jnp.float32),
        pltpu.SemaphoreType.DMA,
        pltpu.SemaphoreType.DMA,
        pltpu.SemaphoreType.DMA,
        pltpu.SemaphoreType.DMA,
    ],
    compiler_params=pltpu.CompilerParams(
        use_tc_tiling_on_sc=False, needs_layout_passes=False
    ),
)
def _moe_route(scores_hbm, masked_hbm, gmask_hbm, buf, gm_v, si_a, si_b, so_a, so_b):
    wid = lax.axis_index("s") * NC + lax.axis_index("c")
    tok_base = wid * TOK_PER_W
    row_base = wid * (TOK_PER_W // 8)
    lanes = lax.iota(jnp.int32, L)
    neg_inf = jnp.full((L,), -jnp.inf, jnp.float32)
    rot = [(lanes + k) & (L - 1) for k in range(L)]
    lane_hi = lanes >> 3
    lane_lo = lanes & 7

    in_sems = (si_a, si_b)
    out_sems = (so_a, so_b)

    def start_in(ci, slot):
        pltpu.async_copy(
            scores_hbm.at[pl.ds(row_base + ci * RPC, RPC)],
            buf.at[pl.ds(slot * RPC, RPC)],
            in_sems[slot],
        )

    def wait_in(slot):
        pltpu.make_async_copy(
            scores_hbm.at[pl.ds(0, RPC)],
            buf.at[pl.ds(slot * RPC, RPC)],
            in_sems[slot],
        ).wait()

    def start_out(ci, slot):
        pltpu.async_copy(
            buf.at[pl.ds(slot * RPC, RPC)],
            masked_hbm.at[pl.ds(row_base + ci * RPC, RPC)],
            out_sems[slot],
        )
        pltpu.async_copy(
            gm_v.at[slot],
            gmask_hbm.at[(tok_base + ci * CHUNK) // 128],
            out_sems[slot],
        )

    def wait_out(slot):
        pltpu.make_async_copy(
            buf.at[pl.ds(slot * RPC, RPC)],
            masked_hbm.at[pl.ds(0, RPC)],
            out_sems[slot],
        ).wait()
        pltpu.make_async_copy(
            gm_v.at[slot],
            gmask_hbm.at[0],
            out_sems[slot],
        ).wait()

    def compute(slot):
        def sub_body(s, carry2):
            rowv = lane_hi + (2 * s + slot * RPC)
            q = [lane_lo, lane_lo + 8]
            m1 = [neg_inf] * N_GROUP
            m2 = [neg_inf] * N_GROUP
            for tile in range(NUM_EXPERTS // L):
                e0 = tile * L
                g = e0 // EPG
                qv = q[e0 >> 7]
                cbase = e0 & 127
                for k in range(L):
                    cv = rot[k] + cbase
                    x = plsc.load_gather(buf, [rowv, qv, cv])
                    t = jnp.minimum(m1[g], x)
                    m1[g] = jnp.maximum(m1[g], x)
                    m2[g] = jnp.maximum(m2[g], t)
            gs = [m1[g] + m2[g] for g in range(N_GROUP)]
            rank = [jnp.zeros((L,), jnp.int32) for _ in range(N_GROUP)]
            one = jnp.ones((L,), jnp.int32)
            zero = jnp.zeros((L,), jnp.int32)
            for g in range(N_GROUP):
                for h in range(g + 1, N_GROUP):
                    ge = gs[g] >= gs[h]
                    rank[h] = rank[h] + jnp.where(ge, one, zero)
                    rank[g] = rank[g] + jnp.where(ge, zero, one)
            sel = [rank[g] < TOPK_GROUP for g in range(N_GROUP)]
            fone = jnp.ones((L,), jnp.float32)
            fzero = jnp.zeros((L,), jnp.float32)
            for g in range(N_GROUP):
                gm_v[slot, g, pl.ds(s * L, L)] = jnp.where(sel[g], fone, fzero)
            for tile in range(NUM_EXPERTS // L):
                e0 = tile * L
                g = e0 // EPG
                nsel = rank[g] >= TOPK_GROUP
                qv = q[e0 >> 7]
                cbase = e0 & 127
                for k in range(L):
                    cv = rot[k] + cbase
                    plsc.store_scatter(buf, [rowv, qv, cv], neg_inf, mask=nsel)
            return carry2

        lax.fori_loop(0, SUB, sub_body, 0)

    start_in(0, 0)

    def pair_body(i, carry):
        c0 = 2 * i

        @pl.when(i > 0)
        def _():
            wait_out(1)

        start_in(c0 + 1, 1)
        wait_in(0)
        compute(0)
        start_out(c0, 0)

        wait_in(1)
        compute(1)

        @pl.when(i < NPAIR - 1)
        def _():
            wait_out(0)
            start_in(c0 + 2, 0)

        start_out(c0 + 1, 1)
        return carry

    lax.fori_loop(0, NPAIR, pair_body, 0)
    wait_out(0)
    wait_out(1)


def kernel(scores):
    s3 = (
        scores.reshape(NROWS, 8, 2, 128)
        .transpose(0, 2, 1, 3)
        .reshape(NROWS, 16, 128)
    )
    masked3, gm3 = _moe_route(s3)
    masked = (
        masked3.reshape(NROWS, 2, 8, 128)
        .transpose(0, 2, 1, 3)
        .reshape(NUM_TOKENS, NUM_EXPERTS)
    )
    gm = gm3.transpose(1, 0, 2).reshape(N_GROUP, NUM_TOKENS).T
    return masked, gm


# speedup vs baseline: 299.4826x; 2.2750x over previous
import functools

import jax
import jax.numpy as jnp
from jax import lax
from jax.experimental import pallas as pl
from jax.experimental.pallas import tpu as pltpu
from jax.experimental.pallas import tpu_sc as plsc

NUM_TOKENS = 32768
NUM_EXPERTS = 256
N_GROUP = 8
EPG = NUM_EXPERTS // N_GROUP
TOPK_GROUP = 4

NC, NS, L = 2, 16, 16
NW = NC * NS
TOK_PER_W = NUM_TOKENS // NW
CHUNK = 128
RPC = CHUNK // 8
NCHUNK = TOK_PER_W // CHUNK
NPAIR = NCHUNK // 2
SUB = CHUNK // L
NROWS = NUM_TOKENS // 8

_mesh = plsc.VectorSubcoreMesh(
    core_axis_name="c", subcore_axis_name="s", num_cores=NC, num_subcores=NS
)


@functools.partial(
    pl.kernel,
    out_type=(
        jax.ShapeDtypeStruct((NROWS, 16, 128), jnp.float32),
        jax.ShapeDtypeStruct((NUM_TOKENS // 128, N_GROUP, 128), jnp.float32),
    ),
    mesh=_mesh,
    scratch_types=[
        pltpu.VMEM((2 * RPC, 16, 128), jnp.float32),
        pltpu.VMEM((2, N_GROUP, CHUNK), jnp.float32),
        pltpu.VMEM((N_GROUP, L), jnp.float32),
        pltpu.VMEM((N_GROUP, L), jnp.int32),
        pltpu.SemaphoreType.DMA,
        pltpu.SemaphoreType.DMA,
        pltpu.SemaphoreType.DMA,
        pltpu.SemaphoreType.DMA,
    ],
    compiler_params=pltpu.CompilerParams(
        use_tc_tiling_on_sc=False, needs_layout_passes=False
    ),
)
def _moe_route(scores_hbm, masked_hbm, gmask_hbm, buf, gm_v, gs_v, sel_v, si_a, si_b, so_a, so_b):
    wid = lax.axis_index("s") * NC + lax.axis_index("c")
    tok_base = wid * TOK_PER_W
    row_base = wid * (TOK_PER_W // 8)
    lanes = lax.iota(jnp.int32, L)
    neg_inf = jnp.full((L,), -jnp.inf, jnp.float32)
    rot = [(lanes + k) & (L - 1) for k in range(L)]
    lane_hi = lanes >> 3
    lane_lo = lanes & 7

    in_sems = (si_a, si_b)
    out_sems = (so_a, so_b)

    def start_in(ci, slot):
        pltpu.async_copy(
            scores_hbm.at[pl.ds(row_base + ci * RPC, RPC)],
            buf.at[pl.ds(slot * RPC, RPC)],
            in_sems[slot],
        )

    def wait_in(slot):
        pltpu.make_async_copy(
            scores_hbm.at[pl.ds(0, RPC)],
            buf.at[pl.ds(slot * RPC, RPC)],
            in_sems[slot],
        ).wait()

    def start_out(ci, slot):
        pltpu.async_copy(
            buf.at[pl.ds(slot * RPC, RPC)],
            masked_hbm.at[pl.ds(row_base + ci * RPC, RPC)],
            out_sems[slot],
        )
        pltpu.async_copy(
            gm_v.at[slot],
            gmask_hbm.at[(tok_base + ci * CHUNK) // 128],
            out_sems[slot],
        )

    def wait_out(slot):
        pltpu.make_async_copy(
            buf.at[pl.ds(slot * RPC, RPC)],
            masked_hbm.at[pl.ds(0, RPC)],
            out_sems[slot],
        ).wait()
        pltpu.make_async_copy(
            gm_v.at[slot],
            gmask_hbm.at[0],
            out_sems[slot],
        ).wait()

    def compute(slot):
        def sub_body(s, carry2):
            rowv = lane_hi + (2 * s + slot * RPC)
            def grp_body(g, carry3):
                qv = lane_lo + ((g & 4) << 1)
                cbase = (g * EPG) & 127
                m1 = neg_inf
                m2 = neg_inf
                for k in range(EPG):
                    cv = rot[k & (L - 1)] + (cbase + (k & ~(L - 1)))
                    x = plsc.load_gather(buf, [rowv, qv, cv])
                    t = jnp.minimum(m1, x)
                    m1 = jnp.maximum(m1, x)
                    m2 = jnp.maximum(m2, t)
                gs_v[g, :] = m1 + m2
                return carry3

            lax.fori_loop(0, N_GROUP, grp_body, 0)
            gs = [gs_v[g, :] for g in range(N_GROUP)]
            rank = [jnp.zeros((L,), jnp.int32) for _ in range(N_GROUP)]
            one = jnp.ones((L,), jnp.int32)
            zero = jnp.zeros((L,), jnp.int32)
            for g in range(N_GROUP):
                for h in range(g + 1, N_GROUP):
                    ge = gs[g] >= gs[h]
                    rank[h] = rank[h] + jnp.where(ge, one, zero)
                    rank[g] = rank[g] + jnp.where(ge, zero, one)
            sel = [rank[g] < TOPK_GROUP for g in range(N_GROUP)]
            fone = jnp.ones((L,), jnp.float32)
            fzero = jnp.zeros((L,), jnp.float32)
            for g in range(N_GROUP):
                gm_v[slot, g, pl.ds(s * L, L)] = jnp.where(sel[g], fone, fzero)
            for g in range(N_GROUP):
                sel_v[g, :] = jnp.where(sel[g], zero, one)
            def grp2_body(g, carry3):
                nsel = sel_v[g, :] != 0
                qv = lane_lo + ((g & 4) << 1)
                cbase = (g * EPG) & 127
                for k in range(EPG):
                    cv = rot[k & (L - 1)] + (cbase + (k & ~(L - 1)))
                    plsc.store_scatter(buf, [rowv, qv, cv], neg_inf, mask=nsel)
                return carry3

            lax.fori_loop(0, N_GROUP, grp2_body, 0)
            return carry2

        lax.fori_loop(0, SUB, sub_body, 0)

    start_in(0, 0)

    def pair_body(i, carry):
        c0 = 2 * i

        @pl.when(i > 0)
        def _():
            wait_out(1)

        start_in(c0 + 1, 1)
        wait_in(0)
        compute(0)
        start_out(c0, 0)

        wait_in(1)
        compute(1)

        @pl.when(i < NPAIR - 1)
        def _():
            wait_out(0)
            start_in(c0 + 2, 0)

        start_out(c0 + 1, 1)
        return carry

    lax.fori_loop(0, NPAIR, pair_body, 0)
    wait_out(0)
    wait_out(1)


def kernel(scores):
    s3 = (
        scores.reshape(NROWS, 8, 2, 128)
        .transpose(0, 2, 1, 3)
        .reshape(NROWS, 16, 128)
    )
    masked3, gm3 = _moe_route(s3)
    masked = (
        masked3.reshape(NROWS, 2, 8, 128)
        .transpose(0, 2, 1, 3)
        .reshape(NUM_TOKENS, NUM_EXPERTS)
    )
    gm = gm3.transpose(1, 0, 2).reshape(N_GROUP, NUM_TOKENS).T
    return masked, gm


# speedup vs baseline: 328.2525x; 1.0961x over previous
import functools

import jax
import jax.numpy as jnp
from jax import lax
from jax.experimental import pallas as pl
from jax.experimental.pallas import tpu as pltpu
from jax.experimental.pallas import tpu_sc as plsc

NUM_TOKENS = 32768
NUM_EXPERTS = 256
N_GROUP = 8
EPG = NUM_EXPERTS // N_GROUP
TOPK_GROUP = 4

NC, NS, L = 2, 16, 16
NW = NC * NS
TOK_PER_W = NUM_TOKENS // NW
CHUNK = 128
RPC = CHUNK // 8
NCHUNK = TOK_PER_W // CHUNK
NPAIR = NCHUNK // 2
SUB = CHUNK // L
NROWS = NUM_TOKENS // 8

_mesh = plsc.VectorSubcoreMesh(
    core_axis_name="c", subcore_axis_name="s", num_cores=NC, num_subcores=NS
)


@functools.partial(
    pl.kernel,
    out_type=(
        jax.ShapeDtypeStruct((NROWS, 16, 128), jnp.float32),
        jax.ShapeDtypeStruct((NUM_TOKENS // 128, N_GROUP, 128), jnp.float32),
    ),
    mesh=_mesh,
    scratch_types=[
        pltpu.VMEM((3 * RPC, 16, 128), jnp.float32),
        pltpu.VMEM((3, N_GROUP, CHUNK), jnp.float32),
        pltpu.VMEM((N_GROUP, L), jnp.float32),
        pltpu.VMEM((N_GROUP, L), jnp.int32),
        pltpu.SemaphoreType.DMA,
        pltpu.SemaphoreType.DMA,
        pltpu.SemaphoreType.DMA,
        pltpu.SemaphoreType.DMA,
        pltpu.SemaphoreType.DMA,
        pltpu.SemaphoreType.DMA,
    ],
    compiler_params=pltpu.CompilerParams(
        use_tc_tiling_on_sc=False, needs_layout_passes=False
    ),
)
def _moe_route(scores_hbm, masked_hbm, gmask_hbm, buf, gm_v, gs_v, sel_v, si_a, si_b, si_c, so_a, so_b, so_c):
    wid = lax.axis_index("s") * NC + lax.axis_index("c")
    tok_base = wid * TOK_PER_W
    row_base = wid * (TOK_PER_W // 8)
    lanes = lax.iota(jnp.int32, L)
    neg_inf = jnp.full((L,), -jnp.inf, jnp.float32)
    rot = [(lanes + k) & (L - 1) for k in range(L)]
    lane_hi = lanes >> 3
    lane_lo = lanes & 7

    in_sems = (si_a, si_b, si_c)
    out_sems = (so_a, so_b, so_c)

    def start_in(ci, slot):
        pltpu.async_copy(
            scores_hbm.at[pl.ds(row_base + ci * RPC, RPC)],
            buf.at[pl.ds(slot * RPC, RPC)],
            in_sems[slot],
        )

    def wait_in(slot):
        pltpu.make_async_copy(
            scores_hbm.at[pl.ds(0, RPC)],
            buf.at[pl.ds(slot * RPC, RPC)],
            in_sems[slot],
        ).wait()

    def start_out(ci, slot):
        pltpu.async_copy(
            buf.at[pl.ds(slot * RPC, RPC)],
            masked_hbm.at[pl.ds(row_base + ci * RPC, RPC)],
            out_sems[slot],
        )
        pltpu.async_copy(
            gm_v.at[slot],
            gmask_hbm.at[(tok_base + ci * CHUNK) // 128],
            out_sems[slot],
        )

    def wait_out(slot):
        pltpu.make_async_copy(
            buf.at[pl.ds(slot * RPC, RPC)],
            masked_hbm.at[pl.ds(0, RPC)],
            out_sems[slot],
        ).wait()
        pltpu.make_async_copy(
            gm_v.at[slot],
            gmask_hbm.at[0],
            out_sems[slot],
        ).wait()

    def compute(slot):
        def sub_body(s, carry2):
            rowv = lane_hi + (2 * s + slot * RPC)
            def grp_body(g, carry3):
                qv = lane_lo + ((g & 4) << 1)
                cbase = (g * EPG) & 127
                m1 = neg_inf
                m2 = neg_inf
                for k in range(EPG):
                    cv = rot[k & (L - 1)] + (cbase + (k & ~(L - 1)))
                    x = plsc.load_gather(buf, [rowv, qv, cv])
                    t = jnp.minimum(m1, x)
                    m1 = jnp.maximum(m1, x)
                    m2 = jnp.maximum(m2, t)
                gs_v[g, :] = m1 + m2
                return carry3

            lax.fori_loop(0, N_GROUP, grp_body, 0)
            gs = [gs_v[g, :] for g in range(N_GROUP)]
            rank = [jnp.zeros((L,), jnp.int32) for _ in range(N_GROUP)]
            one = jnp.ones((L,), jnp.int32)
            zero = jnp.zeros((L,), jnp.int32)
            for g in range(N_GROUP):
                for h in range(g + 1, N_GROUP):
                    ge = gs[g] >= gs[h]
                    rank[h] = rank[h] + jnp.where(ge, one, zero)
                    rank[g] = rank[g] + jnp.where(ge, zero, one)
            sel = [rank[g] < TOPK_GROUP for g in range(N_GROUP)]
            fone = jnp.ones((L,), jnp.float32)
            fzero = jnp.zeros((L,), jnp.float32)
            for g in range(N_GROUP):
                gm_v[slot, g, pl.ds(s * L, L)] = jnp.where(sel[g], fone, fzero)
            for g in range(N_GROUP):
                sel_v[g, :] = jnp.where(sel[g], zero, one)
            def grp2_body(g, carry3):
                nsel = sel_v[g, :] != 0
                qv = lane_lo + ((g & 4) << 1)
                cbase = (g * EPG) & 127
                for k in range(EPG):
                    cv = rot[k & (L - 1)] + (cbase + (k & ~(L - 1)))
                    plsc.store_scatter(buf, [rowv, qv, cv], neg_inf, mask=nsel)
                return carry3

            lax.fori_loop(0, N_GROUP, grp2_body, 0)
            return carry2

        lax.fori_loop(0, SUB, sub_body, 0)

    start_in(0, 0)
    start_in(1, 1)
    for ci in range(NCHUNK):
        slot = ci % 3
        wait_in(slot)
        compute(slot)
        start_out(ci, slot)
        nxt = ci + 2
        if nxt < NCHUNK:
            pslot = nxt % 3
            if ci >= 1:
                wait_out(pslot)
            start_in(nxt, pslot)
    for ci in range(NCHUNK - 3, NCHUNK):
        wait_out(ci % 3)


def kernel(scores):
    s3 = (
        scores.reshape(NROWS, 8, 2, 128)
        .transpose(0, 2, 1, 3)
        .reshape(NROWS, 16, 128)
    )
    masked3, gm3 = _moe_route(s3)
    masked = (
        masked3.reshape(NROWS, 2, 8, 128)
        .transpose(0, 2, 1, 3)
        .reshape(NUM_TOKENS, NUM_EXPERTS)
    )
    gm = gm3.transpose(1, 0, 2).reshape(N_GROUP, NUM_TOKENS).T
    return masked, gm
